# Initial kernel scaffold; baseline (speedup 1.0000x reference)
import jax
import jax.numpy as jnp
from jax.experimental import pallas as pl


def kernel(X, edge_index):
    raise NotImplementedError("write your pallas kernel here")


# speedup vs baseline: 5.9230x; 5.9230x over previous
import functools

import jax
import jax.numpy as jnp
from jax import lax
from jax.experimental import pallas as pl
from jax.experimental.pallas import tpu as pltpu
from jax.experimental.pallas import tpu_sc as plsc

N_NODES = 10000
N_EDGES = 320000
D_FEAT = 128
DH = D_FEAT // 2
NC = 2
NS = 16
CH = 128
EP = -(-N_EDGES // (NS * CH)) * (NS * CH)
NCHUNK = EP // (NS * CH)
ACC_R = 10240
ZR = ACC_R // NS


def _sc_body(x2_h, dst_h, src_h, zr_h, out_h,
             dst_v, src_v, rows_v, acc, sem):
    cid = lax.axis_index("c")
    sid = lax.axis_index("s")

    pltpu.sync_copy(zr_h, acc.at[pl.ds(sid * ZR, ZR)])
    pltpu.sync_copy(dst_h.at[sid], dst_v)
    pltpu.sync_copy(src_h.at[cid, sid], src_v)
    plsc.subcore_barrier()

    def chunk(j, carry):
        pltpu.async_copy(x2_h.at[src_v.at[j]], rows_v, sem).wait()
        pltpu.sync_copy(rows_v, acc.at[dst_v.at[j]], add=True)
        return carry

    lax.fori_loop(0, NCHUNK, chunk, 0)
    plsc.subcore_barrier()

    pltpu.sync_copy(acc.at[pl.ds(sid * ZR, ZR)],
                    out_h.at[cid, pl.ds(sid * ZR, ZR)])


@functools.partial(
    pl.kernel,
    out_type=jax.ShapeDtypeStruct((NC, ACC_R, DH), jnp.float32),
    mesh=plsc.VectorSubcoreMesh(core_axis_name="c", subcore_axis_name="s"),
    compiler_params=pltpu.CompilerParams(use_tc_tiling_on_sc=False),
    scratch_types=[
        pltpu.VMEM((NCHUNK, CH), jnp.int32),
        pltpu.VMEM((NCHUNK, CH), jnp.int32),
        pltpu.VMEM((CH, DH), jnp.float32),
        pltpu.VMEM_SHARED((ACC_R, DH), jnp.float32),
        pltpu.SemaphoreType.DMA,
    ],
)
def _mp_kernel(x2_h, dst_h, src_h, zr_h, out_h,
               dst_v, src_v, rows_v, acc, sem):
    _sc_body(x2_h, dst_h, src_h, zr_h, out_h,
             dst_v, src_v, rows_v, acc, sem)


def kernel(X, edge_index):
    dst = edge_index[0]
    src = edge_index[1]
    pad = EP - N_EDGES
    dstp = jnp.concatenate(
        [dst, jnp.full((pad,), N_NODES, jnp.int32)]).reshape(NS, NCHUNK, CH)
    srcp = jnp.concatenate(
        [src, jnp.zeros((pad,), jnp.int32)]).reshape(NS, NCHUNK, CH)
    src2 = jnp.stack([srcp, srcp + N_NODES])
    x2 = X.reshape(N_NODES, NC, DH).transpose(1, 0, 2).reshape(NC * N_NODES, DH)
    zrows = jnp.zeros((ZR, DH), jnp.float32)
    out = _mp_kernel(x2, dstp, src2, zrows)
    return out[:, :N_NODES].transpose(1, 0, 2).reshape(N_NODES, D_FEAT)


# speedup vs baseline: 7.0705x; 1.1937x over previous
import functools

import jax
import jax.numpy as jnp
from jax import lax
from jax.experimental import pallas as pl
from jax.experimental.pallas import tpu as pltpu
from jax.experimental.pallas import tpu_sc as plsc

N_NODES = 10000
N_EDGES = 320000
D_FEAT = 128
DH = D_FEAT // 2
NC = 2
NS = 16
CH = 128
EP = -(-N_EDGES // (NS * CH)) * (NS * CH)
NCHUNK = EP // (NS * CH)
ACC_R = 10240
ZR = ACC_R // NS


def _sc_body(x2_h, dst_h, src_h, zr_h, out_h,
             dst_v, src_v, rows_v, acc, sem):
    cid = lax.axis_index("c")
    sid = lax.axis_index("s")

    pltpu.sync_copy(zr_h, acc.at[pl.ds(sid * ZR, ZR)])
    pltpu.sync_copy(dst_h.at[sid], dst_v)
    pltpu.sync_copy(src_h.at[cid, sid], src_v)
    plsc.subcore_barrier()

    pltpu.async_copy(x2_h.at[src_v.at[0]], rows_v.at[0], sem)

    def chunk(j, carry):
        b = lax.rem(j, 2)
        pltpu.make_async_copy(x2_h.at[src_v.at[j]], rows_v.at[b], sem).wait()

        @pl.when(j < NCHUNK - 1)
        def _():
            pltpu.async_copy(x2_h.at[src_v.at[j + 1]], rows_v.at[1 - b], sem)

        pltpu.sync_copy(rows_v.at[b], acc.at[dst_v.at[j]], add=True)
        return carry

    lax.fori_loop(0, NCHUNK, chunk, 0)
    plsc.subcore_barrier()

    pltpu.sync_copy(acc.at[pl.ds(sid * ZR, ZR)],
                    out_h.at[cid, pl.ds(sid * ZR, ZR)])


@functools.partial(
    pl.kernel,
    out_type=jax.ShapeDtypeStruct((NC, ACC_R, DH), jnp.float32),
    mesh=plsc.VectorSubcoreMesh(core_axis_name="c", subcore_axis_name="s"),
    compiler_params=pltpu.CompilerParams(use_tc_tiling_on_sc=False),
    scratch_types=[
        pltpu.VMEM((NCHUNK, CH), jnp.int32),
        pltpu.VMEM((NCHUNK, CH), jnp.int32),
        pltpu.VMEM((2, CH, DH), jnp.float32),
        pltpu.VMEM_SHARED((ACC_R, DH), jnp.float32),
        pltpu.SemaphoreType.DMA,
    ],
)
def _mp_kernel(x2_h, dst_h, src_h, zr_h, out_h,
               dst_v, src_v, rows_v, acc, sem):
    _sc_body(x2_h, dst_h, src_h, zr_h, out_h,
             dst_v, src_v, rows_v, acc, sem)


def kernel(X, edge_index):
    dst = edge_index[0]
    src = edge_index[1]
    pad = EP - N_EDGES
    dstp = jnp.concatenate(
        [dst, jnp.full((pad,), N_NODES, jnp.int32)]).reshape(NS, NCHUNK, CH)
    srcp = jnp.concatenate(
        [src, jnp.zeros((pad,), jnp.int32)]).reshape(NS, NCHUNK, CH)
    src2 = jnp.stack([srcp, srcp + N_NODES])
    x2 = X.reshape(N_NODES, NC, DH).transpose(1, 0, 2).reshape(NC * N_NODES, DH)
    zrows = jnp.zeros((ZR, DH), jnp.float32)
    out = _mp_kernel(x2, dstp, src2, zrows)
    return out[:, :N_NODES].transpose(1, 0, 2).reshape(N_NODES, D_FEAT)


# speedup vs baseline: 9.2278x; 1.3051x over previous
import functools

import jax
import jax.numpy as jnp
from jax import lax
from jax.experimental import pallas as pl
from jax.experimental.pallas import tpu as pltpu
from jax.experimental.pallas import tpu_sc as plsc

N_NODES = 10000
N_EDGES = 320000
D_FEAT = 128
DH = D_FEAT // 2
NC = 2
NS = 16
CH = 128
EP = -(-N_EDGES // (NS * CH)) * (NS * CH)
NCHUNK = EP // (NS * CH)
ACC_R = 10240
ZR = ACC_R // NS


NBUF = 6
GAHEAD = 3
SLAG = 2


def _sc_body(x2_h, dst_h, src_h, zr_h, out_h,
             dst_v, src_v, rows_v, acc, gsem, ssem):
    cid = lax.axis_index("c")
    sid = lax.axis_index("s")

    pltpu.sync_copy(zr_h, acc.at[pl.ds(sid * ZR, ZR)])
    pltpu.sync_copy(dst_h.at[sid], dst_v)
    pltpu.sync_copy(src_h.at[cid, sid], src_v)
    plsc.subcore_barrier()

    for k in range(GAHEAD):
        pltpu.async_copy(x2_h.at[src_v.at[k]], rows_v.at[k], gsem)

    def chunk(j, carry):
        b = lax.rem(j, NBUF)
        pltpu.make_async_copy(x2_h.at[src_v.at[j]], rows_v.at[b], gsem).wait()
        pltpu.async_copy(rows_v.at[b], acc.at[dst_v.at[j]], ssem, add=True)

        @pl.when(j >= SLAG)
        def _():
            pltpu.make_async_copy(rows_v.at[b], acc.at[dst_v.at[j]],
                                  ssem).wait()

        @pl.when(j < NCHUNK - GAHEAD)
        def _():
            pltpu.async_copy(x2_h.at[src_v.at[j + GAHEAD]],
                             rows_v.at[lax.rem(j + GAHEAD, NBUF)], gsem)

        return carry

    lax.fori_loop(0, NCHUNK, chunk, 0)
    for _ in range(SLAG):
        pltpu.make_async_copy(rows_v.at[0], acc.at[dst_v.at[0]], ssem).wait()
    plsc.subcore_barrier()

    pltpu.sync_copy(acc.at[pl.ds(sid * ZR, ZR)],
                    out_h.at[cid, pl.ds(sid * ZR, ZR)])


@functools.partial(
    pl.kernel,
    out_type=jax.ShapeDtypeStruct((NC, ACC_R, DH), jnp.float32),
    mesh=plsc.VectorSubcoreMesh(core_axis_name="c", subcore_axis_name="s"),
    compiler_params=pltpu.CompilerParams(use_tc_tiling_on_sc=False),
    scratch_types=[
        pltpu.VMEM((NCHUNK, CH), jnp.int32),
        pltpu.VMEM((NCHUNK, CH), jnp.int32),
        pltpu.VMEM((NBUF, CH, DH), jnp.float32),
        pltpu.VMEM_SHARED((ACC_R, DH), jnp.float32),
        pltpu.SemaphoreType.DMA,
        pltpu.SemaphoreType.DMA,
    ],
)
def _mp_kernel(x2_h, dst_h, src_h, zr_h, out_h,
               dst_v, src_v, rows_v, acc, gsem, ssem):
    _sc_body(x2_h, dst_h, src_h, zr_h, out_h,
             dst_v, src_v, rows_v, acc, gsem, ssem)


def kernel(X, edge_index):
    dst = edge_index[0]
    src = edge_index[1]
    pad = EP - N_EDGES
    dstp = jnp.concatenate(
        [dst, jnp.full((pad,), N_NODES, jnp.int32)]).reshape(NS, NCHUNK, CH)
    srcp = jnp.concatenate(
        [src, jnp.zeros((pad,), jnp.int32)]).reshape(NS, NCHUNK, CH)
    src2 = jnp.stack([srcp, srcp + N_NODES])
    x2 = X.reshape(N_NODES, NC, DH).transpose(1, 0, 2).reshape(NC * N_NODES, DH)
    zrows = jnp.zeros((ZR, DH), jnp.float32)
    out = _mp_kernel(x2, dstp, src2, zrows)
    return out[:, :N_NODES].transpose(1, 0, 2).reshape(N_NODES, D_FEAT)


# speedup vs baseline: 9.3526x; 1.0135x over previous
import functools

import jax
import jax.numpy as jnp
from jax import lax
from jax.experimental import pallas as pl
from jax.experimental.pallas import tpu as pltpu
from jax.experimental.pallas import tpu_sc as plsc

N_NODES = 10000
N_EDGES = 320000
D_FEAT = 128
DH = D_FEAT // 2
NC = 2
NS = 16
CH = 128
EP = -(-N_EDGES // (NS * CH)) * (NS * CH)
NCHUNK = EP // (NS * CH)
ACC_R = 10240
ZR = ACC_R // NS
OR = N_NODES // NS
NBUF = 6
GAHEAD = 3
SLAG = 2


def _sc_body(x2_h, ei_h, zr_h, out_h, dst_v, src_v, rows_v, acc, gsem, ssem):
    cid = lax.axis_index("c")
    sid = lax.axis_index("s")

    pltpu.async_copy(zr_h, acc.at[pl.ds(sid * ZR, ZR)], gsem)
    pltpu.async_copy(ei_h.at[0, sid], dst_v, gsem)
    pltpu.async_copy(ei_h.at[1, sid], src_v, gsem)
    pltpu.make_async_copy(zr_h, acc.at[pl.ds(sid * ZR, ZR)], gsem).wait()
    pltpu.make_async_copy(ei_h.at[0, sid], dst_v, gsem).wait()
    pltpu.make_async_copy(ei_h.at[1, sid], src_v, gsem).wait()

    def remap(j, carry):
        for k in range(CH // 16):
            s = src_v[j, pl.ds(k * 16, 16)]
            src_v[j, pl.ds(k * 16, 16)] = s + s + cid
        return carry

    lax.fori_loop(0, NCHUNK, remap, 0)
    plsc.subcore_barrier()

    for k in range(GAHEAD):
        pltpu.async_copy(x2_h.at[src_v.at[k]], rows_v.at[k], gsem)

    def chunk(j, carry):
        b = lax.rem(j, NBUF)
        pltpu.make_async_copy(x2_h.at[src_v.at[j]], rows_v.at[b], gsem).wait()
        pltpu.async_copy(rows_v.at[b], acc.at[dst_v.at[j]], ssem, add=True)

        @pl.when(j >= SLAG)
        def _():
            pltpu.make_async_copy(rows_v.at[b], acc.at[dst_v.at[j]],
                                  ssem).wait()

        @pl.when(j < NCHUNK - GAHEAD)
        def _():
            pltpu.async_copy(x2_h.at[src_v.at[j + GAHEAD]],
                             rows_v.at[lax.rem(j + GAHEAD, NBUF)], gsem)

        return carry

    lax.fori_loop(0, NCHUNK, chunk, 0)
    for _ in range(SLAG):
        pltpu.make_async_copy(rows_v.at[0], acc.at[dst_v.at[0]], ssem).wait()
    plsc.subcore_barrier()

    pltpu.sync_copy(acc.at[pl.ds(sid * OR, OR)],
                    out_h.at[pl.ds(sid * OR, OR), cid])


@functools.partial(
    pl.kernel,
    out_type=jax.ShapeDtypeStruct((N_NODES, NC, DH), jnp.float32),
    mesh=plsc.VectorSubcoreMesh(core_axis_name="c", subcore_axis_name="s"),
    compiler_params=pltpu.CompilerParams(use_tc_tiling_on_sc=False),
    scratch_types=[
        pltpu.VMEM((NCHUNK, CH), jnp.int32),
        pltpu.VMEM((NCHUNK, CH), jnp.int32),
        pltpu.VMEM((NBUF, CH, DH), jnp.float32),
        pltpu.VMEM_SHARED((ACC_R, DH), jnp.float32),
        pltpu.SemaphoreType.DMA,
        pltpu.SemaphoreType.DMA,
    ],
)
def _mp_kernel(x2_h, ei_h, zr_h, out_h, dst_v, src_v, rows_v, acc,
               gsem, ssem):
    _sc_body(x2_h, ei_h, zr_h, out_h, dst_v, src_v, rows_v, acc, gsem, ssem)


def kernel(X, edge_index):
    pad = jnp.stack([jnp.full((EP - N_EDGES,), N_NODES, jnp.int32),
                     jnp.zeros((EP - N_EDGES,), jnp.int32)])
    eip = jnp.concatenate([edge_index, pad], axis=1)
    eip = eip.reshape(2, NS, NCHUNK, CH)
    x2 = X.reshape(NC * N_NODES, DH)
    zrows = jnp.zeros((ZR, DH), jnp.float32)
    out = _mp_kernel(x2, eip, zrows)
    return out.reshape(N_NODES, D_FEAT)


# speedup vs baseline: 11.5889x; 1.2391x over previous
import functools

import jax
import jax.numpy as jnp
from jax import lax
from jax.experimental import pallas as pl
from jax.experimental.pallas import tpu as pltpu
from jax.experimental.pallas import tpu_sc as plsc

N_NODES = 10000
N_EDGES = 320000
D_FEAT = 128
DH = D_FEAT // 2
NC = 2
NS = 16
CH = 128
EP = -(-N_EDGES // (NS * CH)) * (NS * CH)
NCHUNK = EP // (NS * CH)
ACC_R = 10240
ZR = ACC_R // NS
OR = N_NODES // NS
NBUF = 6
GAHEAD = 3
SLAG = 2


def _sc_body(x2_h, ei_h, zr_h, out_h, dst_v, src_v, rows_v, acc, gsem, ssem):
    cid = lax.axis_index("c")
    sid = lax.axis_index("s")

    pltpu.async_copy(zr_h, acc.at[pl.ds(sid * ZR, ZR)], gsem)
    pltpu.async_copy(ei_h.at[0, sid], dst_v, gsem)
    pltpu.async_copy(ei_h.at[1, sid], src_v, gsem)
    pltpu.make_async_copy(zr_h, acc.at[pl.ds(sid * ZR, ZR)], gsem).wait()
    pltpu.make_async_copy(ei_h.at[0, sid], dst_v, gsem).wait()
    pltpu.make_async_copy(ei_h.at[1, sid], src_v, gsem).wait()

    plsc.subcore_barrier()

    def remap(j):
        for k in range(CH // 16):
            s = src_v[j, pl.ds(k * 16, 16)]
            src_v[j, pl.ds(k * 16, 16)] = s + s + cid

    for k in range(GAHEAD):
        remap(k)
        pltpu.async_copy(x2_h.at[src_v.at[k]], rows_v.at[k], gsem)

    def chunk(j, carry):
        @pl.when(j < NCHUNK - GAHEAD)
        def _():
            remap(j + GAHEAD)

        b = lax.rem(j, NBUF)
        pltpu.make_async_copy(x2_h.at[src_v.at[j]], rows_v.at[b], gsem).wait()
        pltpu.async_copy(rows_v.at[b], acc.at[dst_v.at[j]], ssem, add=True)

        @pl.when(j >= SLAG)
        def _():
            pltpu.make_async_copy(rows_v.at[b], acc.at[dst_v.at[j]],
                                  ssem).wait()

        @pl.when(j < NCHUNK - GAHEAD)
        def _():
            pltpu.async_copy(x2_h.at[src_v.at[j + GAHEAD]],
                             rows_v.at[lax.rem(j + GAHEAD, NBUF)], gsem)

        return carry

    lax.fori_loop(0, NCHUNK, chunk, 0)
    for _ in range(SLAG):
        pltpu.make_async_copy(rows_v.at[0], acc.at[dst_v.at[0]], ssem).wait()
    plsc.subcore_barrier()

    pltpu.sync_copy(acc.at[pl.ds(sid * OR, OR)],
                    out_h.at[pl.ds(sid * OR, OR), pl.ds(cid * DH, DH)])


@functools.partial(
    pl.kernel,
    out_type=jax.ShapeDtypeStruct((N_NODES, D_FEAT), jnp.float32),
    mesh=plsc.VectorSubcoreMesh(core_axis_name="c", subcore_axis_name="s"),
    compiler_params=pltpu.CompilerParams(use_tc_tiling_on_sc=False),
    scratch_types=[
        pltpu.VMEM((NCHUNK, CH), jnp.int32),
        pltpu.VMEM((NCHUNK, CH), jnp.int32),
        pltpu.VMEM((NBUF, CH, DH), jnp.float32),
        pltpu.VMEM_SHARED((ACC_R, DH), jnp.float32),
        pltpu.SemaphoreType.DMA,
        pltpu.SemaphoreType.DMA,
    ],
)
def _mp_kernel(x2_h, ei_h, zr_h, out_h, dst_v, src_v, rows_v, acc,
               gsem, ssem):
    _sc_body(x2_h, ei_h, zr_h, out_h, dst_v, src_v, rows_v, acc, gsem, ssem)


def kernel(X, edge_index):
    pad = jnp.stack([jnp.full((EP - N_EDGES,), N_NODES, jnp.int32),
                     jnp.zeros((EP - N_EDGES,), jnp.int32)])
    eip = jnp.concatenate([edge_index, pad], axis=1)
    eip = eip.reshape(2, NS, NCHUNK, CH)
    x2 = X.reshape(NC * N_NODES, DH)
    zrows = jnp.zeros((ZR, DH), jnp.float32)
    return _mp_kernel(x2, eip, zrows)


# speedup vs baseline: 11.6444x; 1.0048x over previous
import functools

import jax
import jax.numpy as jnp
from jax import lax
from jax.experimental import pallas as pl
from jax.experimental.pallas import tpu as pltpu
from jax.experimental.pallas import tpu_sc as plsc

N_NODES = 10000
N_EDGES = 320000
D_FEAT = 128
DH = D_FEAT // 2
NC = 2
NS = 16
CH = 128
EP = -(-N_EDGES // (NS * CH)) * (NS * CH)
NCHUNK = EP // (NS * CH)
ACC_R = 10240
ZR = ACC_R // NS
OR = N_NODES // NS
NBUF = 6
GAHEAD = 4
SLAG = 2


def _sc_body(x2_h, ei_h, zr_h, out_h, dst_v, src_v, rows_v, acc, gsem, ssem):
    cid = lax.axis_index("c")
    sid = lax.axis_index("s")

    pltpu.async_copy(zr_h, acc.at[pl.ds(sid * ZR, ZR)], gsem)
    pltpu.async_copy(ei_h.at[0, sid], dst_v, gsem)
    pltpu.async_copy(ei_h.at[1, sid], src_v, gsem)
    pltpu.make_async_copy(zr_h, acc.at[pl.ds(sid * ZR, ZR)], gsem).wait()
    pltpu.make_async_copy(ei_h.at[0, sid], dst_v, gsem).wait()
    pltpu.make_async_copy(ei_h.at[1, sid], src_v, gsem).wait()

    plsc.subcore_barrier()

    def remap(j):
        for k in range(CH // 16):
            s = src_v[j, pl.ds(k * 16, 16)]
            src_v[j, pl.ds(k * 16, 16)] = s + s + cid

    for k in range(GAHEAD):
        remap(k)
        pltpu.async_copy(x2_h.at[src_v.at[k]], rows_v.at[k], gsem)

    def chunk(j, carry):
        @pl.when(j < NCHUNK - GAHEAD)
        def _():
            remap(j + GAHEAD)

        b = lax.rem(j, NBUF)
        pltpu.make_async_copy(x2_h.at[src_v.at[j]], rows_v.at[b], gsem).wait()
        pltpu.async_copy(rows_v.at[b], acc.at[dst_v.at[j]], ssem, add=True)

        @pl.when(j >= SLAG)
        def _():
            pltpu.make_async_copy(rows_v.at[b], acc.at[dst_v.at[j]],
                                  ssem).wait()

        @pl.when(j < NCHUNK - GAHEAD)
        def _():
            pltpu.async_copy(x2_h.at[src_v.at[j + GAHEAD]],
                             rows_v.at[lax.rem(j + GAHEAD, NBUF)], gsem)

        return carry

    lax.fori_loop(0, NCHUNK, chunk, 0)
    for _ in range(SLAG):
        pltpu.make_async_copy(rows_v.at[0], acc.at[dst_v.at[0]], ssem).wait()
    plsc.subcore_barrier()

    pltpu.sync_copy(acc.at[pl.ds(sid * OR, OR)],
                    out_h.at[pl.ds(sid * OR, OR), pl.ds(cid * DH, DH)])


@functools.partial(
    pl.kernel,
    out_type=jax.ShapeDtypeStruct((N_NODES, D_FEAT), jnp.float32),
    mesh=plsc.VectorSubcoreMesh(core_axis_name="c", subcore_axis_name="s"),
    compiler_params=pltpu.CompilerParams(use_tc_tiling_on_sc=False),
    scratch_types=[
        pltpu.VMEM((NCHUNK, CH), jnp.int32),
        pltpu.VMEM((NCHUNK, CH), jnp.int32),
        pltpu.VMEM((NBUF, CH, DH), jnp.float32),
        pltpu.VMEM_SHARED((ACC_R, DH), jnp.float32),
        pltpu.SemaphoreType.DMA,
        pltpu.SemaphoreType.DMA,
    ],
)
def _mp_kernel(x2_h, ei_h, zr_h, out_h, dst_v, src_v, rows_v, acc,
               gsem, ssem):
    _sc_body(x2_h, ei_h, zr_h, out_h, dst_v, src_v, rows_v, acc, gsem, ssem)


def kernel(X, edge_index):
    pad = jnp.stack([jnp.full((EP - N_EDGES,), N_NODES, jnp.int32),
                     jnp.zeros((EP - N_EDGES,), jnp.int32)])
    eip = jnp.concatenate([edge_index, pad], axis=1)
    eip = eip.reshape(2, NS, NCHUNK, CH)
    x2 = X.reshape(NC * N_NODES, DH)
    zrows = jnp.zeros((ZR, DH), jnp.float32)
    return _mp_kernel(x2, eip, zrows)
